# baseline (speedup 1.0000x reference)
import functools

import jax
import jax.numpy as jnp
from jax import lax
from jax.experimental import pallas as pl
from jax.experimental.pallas import tpu as pltpu
from jax.experimental.pallas import tpu_sc as plsc

_NC = 2
_NS = 16
_NW = _NC * _NS


def _make_sc_kernel(s, b, d, rows_per_w):
    mesh = plsc.VectorSubcoreMesh(
        core_axis_name="c", subcore_axis_name="s",
        num_cores=_NC, num_subcores=_NS)

    @functools.partial(
        pl.kernel,
        out_type=jax.ShapeDtypeStruct((s, b * d), jnp.float32),
        mesh=mesh,
        scratch_types=[pltpu.SemaphoreType.DMA],
    )
    def sc_copy(table_hbm, out_hbm, sem):
        wid = lax.axis_index("s") * _NC + lax.axis_index("c")
        base = wid * rows_per_w
        copies = []
        for j in range(b):
            copies.append(pltpu.async_copy(
                table_hbm.at[pl.ds(base, rows_per_w)],
                out_hbm.at[pl.ds(base, rows_per_w), pl.ds(j * d, d)],
                sem))
        for cp in copies:
            cp.wait()

    return sc_copy


def kernel(x, table):
    s = x.shape[0]
    b = x.shape[1]
    d = table.shape[1]
    rows_per_w = s // _NW
    out2d = _make_sc_kernel(s, b, d, rows_per_w)(table)
    return out2d.reshape(s, b, d)


# speedup vs baseline: 16.8076x; 16.8076x over previous
import functools

import jax
import jax.numpy as jnp
from jax import lax
from jax.experimental import pallas as pl
from jax.experimental.pallas import tpu as pltpu
from jax.experimental.pallas import tpu_sc as plsc

_NC = 2
_NS = 16
_NW = _NC * _NS
_NBUF = 3


def _make_sc_kernel(s, b, d, rows_per_w, ch):
    nchunk = rows_per_w // ch
    nbuf = min(_NBUF, nchunk)
    mesh = plsc.VectorSubcoreMesh(
        core_axis_name="c", subcore_axis_name="s",
        num_cores=_NC, num_subcores=_NS)

    @functools.partial(
        pl.kernel,
        out_type=jax.ShapeDtypeStruct((s, b * d), jnp.float32),
        mesh=mesh,
        scratch_types=(
            [pltpu.VMEM((ch, d), jnp.float32)] * nbuf
            + [pltpu.SemaphoreType.DMA] * (2 * nbuf)
        ),
    )
    def sc_copy(table_hbm, out_hbm, *scratch):
        bufs = scratch[:nbuf]
        sem_in = scratch[nbuf:2 * nbuf]
        sem_out = scratch[2 * nbuf:]
        wid = lax.axis_index("s") * _NC + lax.axis_index("c")
        base = wid * rows_per_w

        in_copies = [None] * nchunk
        out_copies = [None] * nchunk
        drained = [False] * nchunk

        def start_in(c):
            in_copies[c] = pltpu.async_copy(
                table_hbm.at[pl.ds(base + c * ch, ch)],
                bufs[c % nbuf], sem_in[c % nbuf])

        def drain_outs(c):
            if 0 <= c < nchunk and out_copies[c] is not None and not drained[c]:
                for cp in out_copies[c]:
                    cp.wait()
                drained[c] = True

        start_in(0)
        for c in range(nchunk):
            nxt = c + 1
            if nxt < nchunk:
                drain_outs(nxt - nbuf)
                start_in(nxt)
            in_copies[c].wait()
            row0 = base + c * ch
            out_copies[c] = [
                pltpu.async_copy(
                    bufs[c % nbuf],
                    out_hbm.at[pl.ds(row0, ch), pl.ds(j * d, d)],
                    sem_out[c % nbuf])
                for j in range(b)
            ]
        for c in range(nchunk):
            drain_outs(c)

    return sc_copy


def kernel(x, table):
    s = x.shape[0]
    b = x.shape[1]
    d = table.shape[1]
    rows_per_w = s // _NW
    ch = min(32, rows_per_w)
    out2d = _make_sc_kernel(s, b, d, rows_per_w, ch)(table)
    return out2d.reshape(s, b, d)


# speedup vs baseline: 16.8981x; 1.0054x over previous
import functools

import jax
import jax.numpy as jnp
from jax import lax
from jax.experimental import pallas as pl
from jax.experimental.pallas import tpu as pltpu
from jax.experimental.pallas import tpu_sc as plsc

_NC = 2
_NS = 16

_N_STREAM = 88
_STREAM_CH = 32
_SPMEM_CH = 256


def _chunk_sizes(total, ch):
    sizes = [ch] * (total // ch)
    if total % ch:
        sizes.append(total % ch)
    return sizes


def _copy_pipeline(src_hbm, out_hbm, bufs, sem_in, sem_out, base, sizes, b, d):
    nbuf = len(bufs)
    nchunk = len(sizes)
    offs = [base] * nchunk
    for c in range(1, nchunk):
        offs[c] = offs[c - 1] + sizes[c - 1]
    in_copies = [None] * nchunk
    out_copies = [None] * nchunk
    drained = [False] * nchunk

    def start_in(c):
        in_copies[c] = pltpu.async_copy(
            src_hbm.at[pl.ds(offs[c], sizes[c])],
            bufs[c % nbuf].at[pl.ds(0, sizes[c])], sem_in[c % nbuf])

    def drain_outs(c):
        if 0 <= c < nchunk and out_copies[c] is not None and not drained[c]:
            for cp in out_copies[c]:
                cp.wait()
            drained[c] = True

    start_in(0)
    for c in range(nchunk):
        nxt = c + 1
        if nxt < nchunk:
            drain_outs(nxt - nbuf)
            start_in(nxt)
        in_copies[c].wait()
        out_copies[c] = [
            pltpu.async_copy(
                bufs[c % nbuf].at[pl.ds(0, sizes[c])],
                out_hbm.at[pl.ds(offs[c], sizes[c]), pl.ds(j * d, d)],
                sem_out[c % nbuf])
            for j in range(b)
        ]
    for c in range(nchunk):
        drain_outs(c)


def _make_sc_kernel(s, b, d):
    half = s // _NC
    sp_rows = half - (_NS - 1) * _N_STREAM
    stream_sizes = _chunk_sizes(_N_STREAM, _STREAM_CH)
    spmem_sizes = _chunk_sizes(sp_rows, _SPMEM_CH)
    n_tile_buf = 3
    n_sh_buf = 2
    mesh = plsc.VectorSubcoreMesh(
        core_axis_name="c", subcore_axis_name="s",
        num_cores=_NC, num_subcores=_NS)

    @functools.partial(
        pl.kernel,
        out_type=jax.ShapeDtypeStruct((s, b * d), jnp.float32),
        mesh=mesh,
        scratch_types=(
            [pltpu.VMEM((_STREAM_CH, d), jnp.float32)] * n_tile_buf
            + [pltpu.VMEM_SHARED((_SPMEM_CH, d), jnp.float32)] * n_sh_buf
            + [pltpu.SemaphoreType.DMA] * (2 * max(n_tile_buf, n_sh_buf))
        ),
    )
    def sc_copy(table_hbm, out_hbm, *scratch):
        tile_bufs = scratch[:n_tile_buf]
        sh_bufs = scratch[n_tile_buf:n_tile_buf + n_sh_buf]
        sems = scratch[n_tile_buf + n_sh_buf:]
        nsem = len(sems) // 2
        sem_in = sems[:nsem]
        sem_out = sems[nsem:]
        cid = lax.axis_index("c")
        sid = lax.axis_index("s")
        core_base = cid * half

        @pl.when(sid == 0)
        def _spmem_driver():
            _copy_pipeline(table_hbm, out_hbm,
                           sh_bufs, sem_in[:n_sh_buf], sem_out[:n_sh_buf],
                           core_base, spmem_sizes, b, d)

        @pl.when(sid != 0)
        def _stream_worker():
            base = core_base + sp_rows + (sid - 1) * _N_STREAM
            _copy_pipeline(table_hbm, out_hbm,
                           tile_bufs, sem_in, sem_out,
                           base, stream_sizes, b, d)

    return sc_copy


def kernel(x, table):
    s = x.shape[0]
    b = x.shape[1]
    d = table.shape[1]
    out2d = _make_sc_kernel(s, b, d)(table)
    return out2d.reshape(s, b, d)
